# baseline (speedup 1.0000x reference)
import jax
import jax.numpy as jnp
from jax.experimental import pallas as pl

_NEW = 9
_C = 6 * _NEW * _NEW


def _build_affine(H_z, H_x, qubit_rows, qubit_cols, qubit_src_idx,
                  z_stab_rows, z_stab_cols, z_stab_src_idx,
                  x_stab_rows, x_stab_cols, x_stab_src_idx, dtype):
    n_z = H_z.shape[0]
    n_x = H_x.shape[0]
    nsyn = n_z + n_x
    posz = 2 * 81 + z_stab_rows * _NEW + z_stab_cols
    posx = 3 * 81 + x_stab_rows * _NEW + x_stab_cols
    posq4 = 4 * 81 + qubit_rows * _NEW + qubit_cols
    posq5 = 5 * 81 + qubit_rows * _NEW + qubit_cols
    M = jnp.zeros((nsyn, _C), dtype=dtype)
    M = M.at[z_stab_src_idx, posz].set(2.0)
    M = M.at[n_z + x_stab_src_idx, posx].set(2.0)
    M = M.at[:n_z, posq4].set(H_z[:, qubit_src_idx].astype(dtype) / 4.0)
    M = M.at[n_z:, posq5].set(H_x[:, qubit_src_idx].astype(dtype) / 4.0)
    bias = jnp.zeros((_C,), dtype=dtype)
    bias = bias.at[posz].set(-1.0)
    bias = bias.at[posx].set(-1.0)
    return M, bias


def _body(s_ref, m_ref, b_ref, o_ref):
    o_ref[...] = (
        jnp.dot(s_ref[...], m_ref[...], preferred_element_type=jnp.float32)
        + b_ref[...]
    )


def kernel(syndrome, H_z, H_x, qubit_rows, qubit_cols, qubit_src_idx,
           z_stab_rows, z_stab_cols, z_stab_src_idx,
           x_stab_rows, x_stab_cols, x_stab_src_idx):
    B = syndrome.shape[0]
    nsyn = H_z.shape[0] + H_x.shape[0]
    M, bias = _build_affine(H_z, H_x, qubit_rows, qubit_cols, qubit_src_idx,
                            z_stab_rows, z_stab_cols, z_stab_src_idx,
                            x_stab_rows, x_stab_cols, x_stab_src_idx,
                            syndrome.dtype)
    BLK = 2048
    out = pl.pallas_call(
        _body,
        grid=(B // BLK,),
        in_specs=[
            pl.BlockSpec((BLK, nsyn), lambda i: (i, 0)),
            pl.BlockSpec((nsyn, _C), lambda i: (0, 0)),
            pl.BlockSpec((1, _C), lambda i: (0, 0)),
        ],
        out_specs=pl.BlockSpec((BLK, _C), lambda i: (i, 0)),
        out_shape=jax.ShapeDtypeStruct((B, _C), syndrome.dtype),
    )(syndrome, M, bias[None, :])
    return out.reshape(B, 6, _NEW, _NEW)


# speedup vs baseline: 1.2875x; 1.2875x over previous
import jax
import jax.numpy as jnp
from jax.experimental import pallas as pl

_NEW = 9


def _build_mt(H_z, H_x, qubit_rows, qubit_cols, qubit_src_idx,
              z_stab_rows, z_stab_cols, z_stab_src_idx,
              x_stab_rows, x_stab_cols, x_stab_src_idx, dtype):
    n_z = H_z.shape[0]
    nsyn = n_z + H_x.shape[0]
    MT = jnp.zeros((_NEW, _NEW, 6, nsyn + 1), dtype=dtype)
    MT = MT.at[z_stab_rows, z_stab_cols, 2, z_stab_src_idx].set(2.0)
    MT = MT.at[z_stab_rows, z_stab_cols, 2, nsyn].set(-1.0)
    MT = MT.at[x_stab_rows, x_stab_cols, 3, n_z + x_stab_src_idx].set(2.0)
    MT = MT.at[x_stab_rows, x_stab_cols, 3, nsyn].set(-1.0)
    MT = MT.at[qubit_rows, qubit_cols, 4, :n_z].set(
        H_z[:, qubit_src_idx].T.astype(dtype) / 4.0)
    MT = MT.at[qubit_rows, qubit_cols, 5, n_z:nsyn].set(
        H_x[:, qubit_src_idx].T.astype(dtype) / 4.0)
    return MT


def _body(s_ref, mt_ref, o_ref):
    s = s_ref[...]
    for r in range(_NEW):
        for c in range(_NEW):
            o_ref[r, c] = jax.lax.dot_general(
                mt_ref[r, c], s, (((1,), (0,)), ((), ())),
                preferred_element_type=jnp.float32)


def kernel(syndrome, H_z, H_x, qubit_rows, qubit_cols, qubit_src_idx,
           z_stab_rows, z_stab_cols, z_stab_src_idx,
           x_stab_rows, x_stab_cols, x_stab_src_idx):
    B = syndrome.shape[0]
    nsyn = H_z.shape[0] + H_x.shape[0]
    MT = _build_mt(H_z, H_x, qubit_rows, qubit_cols, qubit_src_idx,
                   z_stab_rows, z_stab_cols, z_stab_src_idx,
                   x_stab_rows, x_stab_cols, x_stab_src_idx,
                   syndrome.dtype)
    sA = jnp.concatenate(
        [syndrome.T, jnp.ones((1, B), dtype=syndrome.dtype)], axis=0)
    BLK = 2048
    outT = pl.pallas_call(
        _body,
        grid=(B // BLK,),
        in_specs=[
            pl.BlockSpec((nsyn + 1, BLK), lambda i: (0, i)),
            pl.BlockSpec((_NEW, _NEW, 6, nsyn + 1), lambda i: (0, 0, 0, 0)),
        ],
        out_specs=pl.BlockSpec((_NEW, _NEW, 6, BLK), lambda i: (0, 0, 0, i)),
        out_shape=jax.ShapeDtypeStruct((_NEW, _NEW, 6, B), syndrome.dtype),
    )(sA, MT)
    return jnp.transpose(outT, (3, 2, 0, 1))
